# baseline (speedup 1.0000x reference)
import functools
import math

import jax
import jax.numpy as jnp
from jax import lax
from jax.experimental import pallas as pl
from jax.experimental.pallas import tpu as pltpu
from jax.experimental.pallas import tpu_sc as plsc

_D = 64
_DP = 128
_SCALE = math.sqrt(_D)
_NC = 2
_NS = 16
_L = 16
_NW = _NC * _NS
_CHUNK = 80
_T = 20
_BC = _CHUNK // _T


@functools.lru_cache(maxsize=None)
def _build(BATCH: int):
    B = BATCH * _T
    BPW = B // _NW
    NCH = BPW // _CHUNK
    BW = BATCH // _NW

    mesh = plsc.VectorSubcoreMesh(core_axis_name="c", subcore_axis_name="s")

    @functools.partial(
        pl.kernel,
        mesh=mesh,
        out_type=jax.ShapeDtypeStruct((BATCH, _T, _D), jnp.float32),
        compiler_params=pltpu.CompilerParams(use_tc_tiling_on_sc=True),
        scratch_types=[
            pltpu.VMEM((NCH, _CHUNK), jnp.int32),
            pltpu.VMEM((_CHUNK, _DP), jnp.float32),
            pltpu.VMEM((_CHUNK, _DP), jnp.float32),
            pltpu.VMEM((_BC, _T, _D), jnp.float32),
            pltpu.VMEM((_BC, _T, _D), jnp.float32),
            pltpu.SemaphoreType.DMA,
            pltpu.SemaphoreType.DMA,
            pltpu.SemaphoreType.DMA,
            pltpu.SemaphoreType.DMA,
        ],
    )
    def emb(idx_hbm, tab_hbm, out_hbm, idx_v, ina, inb, outa, outb,
            gsa, gsb, osa, osb):
        wid = lax.axis_index("s") * _NC + lax.axis_index("c")
        bbase = wid * BW
        pltpu.sync_copy(idx_hbm.at[wid], idx_v)

        ins = (ina, inb)
        outs = (outa, outb)
        gsems = (gsa, gsb)
        osems = (osa, osb)

        def fire_gather(c, j):
            pltpu.async_copy(tab_hbm.at[idx_v.at[c]], ins[j], gsems[j])

        def compute(j):
            for r in range(_CHUNK):
                bl, tt = divmod(r, _T)
                for jj in range(_D // _L):
                    sl = pl.ds(jj * _L, _L)
                    outs[j][bl, tt, sl] = ins[j][r, sl] * _SCALE

        fire_gather(0, 0)
        fire_gather(1, 1)

        def body(k, carry):
            for j in range(2):
                c = 2 * k + j
                pltpu.make_async_copy(
                    tab_hbm.at[idx_v.at[0]], ins[j], gsems[j]).wait()
                @pl.when(k > 0)
                def _():
                    pltpu.make_async_copy(
                        outs[j], out_hbm.at[pl.ds(bbase, _BC)], osems[j]
                    ).wait()
                compute(j)
                pltpu.async_copy(
                    outs[j],
                    out_hbm.at[pl.ds(bbase + c * _BC, _BC)],
                    osems[j],
                )
                @pl.when(c + 2 < NCH)
                def _():
                    pltpu.async_copy(
                        tab_hbm.at[idx_v.at[c + 2]], ins[j], gsems[j])
            return carry

        lax.fori_loop(0, NCH // 2, body, 0)
        for j in range(2):
            pltpu.make_async_copy(
                outs[j], out_hbm.at[pl.ds(bbase, _BC)], osems[j]).wait()

    return emb


def kernel(token, lookup_table):
    BATCH = token.shape[0]
    B = BATCH * _T
    tab128 = jnp.pad(lookup_table.astype(jnp.float32), ((0, 0), (0, _DP - _D)))
    idx = token.reshape(-1).reshape(_NW, (B // _NW) // _CHUNK, _CHUNK)
    idx = idx.astype(jnp.int32)
    return _build(BATCH)(idx, tab128)
